# Initial kernel scaffold; baseline (speedup 1.0000x reference)
import jax
import jax.numpy as jnp
from jax.experimental import pallas as pl


def kernel(x, pos, batch, W1, b1, W2, b2):
    raise NotImplementedError("write your pallas kernel here")


# speedup vs baseline: 4.3655x; 4.3655x over previous
import functools

import jax
import jax.numpy as jnp
from jax import lax
from jax.experimental import pallas as pl
from jax.experimental.pallas import tpu as pltpu

N = 10000
S = 2500
D = 128
K = 32
RADIUS = 0.25

FPS_ROWS = 80
OUT_ROWS = 24
CHUNK = 1024


def _fps_body(px_ref, py_ref, pz_ref, idx_ref, ox_ref, oy_ref, oz_ref):
    px = px_ref[:]
    py = py_ref[:]
    pz = pz_ref[:]
    shape = (FPS_ROWS, 128)
    lin = (lax.broadcasted_iota(jnp.int32, shape, 0) * 128
           + lax.broadcasted_iota(jnp.int32, shape, 1))
    valid = lin < N
    dists0 = jnp.where(valid, jnp.inf, -1.0).astype(jnp.float32)

    cshape = (8, 128)
    clin = (lax.broadcasted_iota(jnp.int32, cshape, 0) * 128
            + lax.broadcasted_iota(jnp.int32, cshape, 1))
    czero_i = jnp.zeros(cshape, jnp.int32)
    czero_f = jnp.zeros(cshape, jnp.float32)

    def body(i, carry):
        dists, cidx, cx, cy, cz = carry
        m = jnp.max(dists)
        sel = dists == m
        nxt = jnp.min(jnp.where(sel, lin, jnp.int32(2**30)))
        one = lin == nxt
        qx = jnp.sum(jnp.where(one, px, 0.0))
        qy = jnp.sum(jnp.where(one, py, 0.0))
        qz = jnp.sum(jnp.where(one, pz, 0.0))
        dx = px - qx
        dy = py - qy
        dz = pz - qz
        dnew = (dx * dx + dy * dy) + dz * dz
        dists = jnp.minimum(dists, dnew)
        slot = lax.rem(i, CHUNK)
        hit = clin == slot
        cidx = jnp.where(hit, nxt, cidx)
        cx = jnp.where(hit, qx, cx)
        cy = jnp.where(hit, qy, cy)
        cz = jnp.where(hit, qz, cz)

        @pl.when((slot == CHUNK - 1) | (i == S - 1))
        def _flush():
            base = lax.div(i, CHUNK) * 8
            idx_ref[pl.ds(base, 8), :] = cidx
            ox_ref[pl.ds(base, 8), :] = cx
            oy_ref[pl.ds(base, 8), :] = cy
            oz_ref[pl.ds(base, 8), :] = cz

        return dists, cidx, cx, cy, cz

    lax.fori_loop(0, S, body, (dists0, czero_i, czero_f, czero_f, czero_f))


def _run_fps(pos, interpret=False):
    pad = FPS_ROWS * 128 - N
    big = jnp.float32(1e9)
    coords = []
    for c in range(3):
        col = jnp.concatenate([pos[:, c], jnp.full((pad,), big, jnp.float32)])
        coords.append(col.reshape(FPS_ROWS, 128))
    out_shapes = (
        jax.ShapeDtypeStruct((OUT_ROWS, 128), jnp.int32),
        jax.ShapeDtypeStruct((OUT_ROWS, 128), jnp.float32),
        jax.ShapeDtypeStruct((OUT_ROWS, 128), jnp.float32),
        jax.ShapeDtypeStruct((OUT_ROWS, 128), jnp.float32),
    )
    idx_o, ox, oy, oz = pl.pallas_call(
        _fps_body,
        out_shape=out_shapes,
        interpret=interpret,
    )(*coords)
    fps_idx = idx_o.reshape(-1)[:S]
    pos_sub = jnp.stack(
        [ox.reshape(-1)[:S], oy.reshape(-1)[:S], oz.reshape(-1)[:S]], axis=1)
    return fps_idx, pos_sub


def kernel(x, pos, batch, W1, b1, W2, b2):
    fps_idx, pos_sub = _run_fps(pos)
    d2 = jnp.sum((pos_sub[:, None, :] - pos[None, :, :]) ** 2, axis=-1)
    neg = jnp.where(d2 <= RADIUS * RADIUS, -d2, -jnp.inf)
    vals, nbr = jax.lax.top_k(neg, K)
    valid = vals > -jnp.inf
    rel = pos[nbr] - pos_sub[:, None, :]
    feat = jnp.concatenate([x[nbr], rel], axis=-1)
    h = jnp.maximum(jnp.einsum('skd,df->skf', feat, W1) + b1, 0.0)
    h = jnp.einsum('skd,df->skf', h, W2) + b2
    h = jnp.where(valid[:, :, None], h, -jnp.inf)
    out = jnp.max(h, axis=1)
    out = jnp.where(jnp.isfinite(out), out, 0.0)
    return (out, pos_sub, batch[fps_idx])


# speedup vs baseline: 4.3959x; 1.0070x over previous
import functools

import jax
import jax.numpy as jnp
from jax import lax
from jax.experimental import pallas as pl
from jax.experimental.pallas import tpu as pltpu

N = 10000
S = 2500
D = 128
K = 32
RADIUS = 0.25

FPS_ROWS = 80
OUT_ROWS = 24
CHUNK = 1024


def _fps_body(px_ref, py_ref, pz_ref, idx_ref, ox_ref, oy_ref, oz_ref):
    px = px_ref[:]
    py = py_ref[:]
    pz = pz_ref[:]
    shape = (FPS_ROWS, 128)
    lin = (lax.broadcasted_iota(jnp.int32, shape, 0) * 128
           + lax.broadcasted_iota(jnp.int32, shape, 1))
    valid = lin < N
    dists0 = jnp.where(valid, jnp.inf, -1.0).astype(jnp.float32)

    cshape = (8, 128)
    clin = (lax.broadcasted_iota(jnp.int32, cshape, 0) * 128
            + lax.broadcasted_iota(jnp.int32, cshape, 1))
    czero_i = jnp.zeros(cshape, jnp.int32)
    czero_f = jnp.zeros(cshape, jnp.float32)

    def body(i, carry):
        dists, cidx, cx, cy, cz = carry
        m = jnp.max(dists)
        sel = dists == m
        nxt = jnp.min(jnp.where(sel, lin, jnp.int32(2**30)))
        one = lin == nxt
        qx = jnp.sum(jnp.where(one, px, 0.0))
        qy = jnp.sum(jnp.where(one, py, 0.0))
        qz = jnp.sum(jnp.where(one, pz, 0.0))
        dx = px - qx
        dy = py - qy
        dz = pz - qz
        dnew = (dx * dx + dy * dy) + dz * dz
        dists = jnp.minimum(dists, dnew)
        slot = lax.rem(i, CHUNK)
        hit = clin == slot
        cidx = jnp.where(hit, nxt, cidx)
        cx = jnp.where(hit, qx, cx)
        cy = jnp.where(hit, qy, cy)
        cz = jnp.where(hit, qz, cz)

        @pl.when((slot == CHUNK - 1) | (i == S - 1))
        def _flush():
            base = lax.div(i, CHUNK) * 8
            idx_ref[pl.ds(base, 8), :] = cidx
            ox_ref[pl.ds(base, 8), :] = cx
            oy_ref[pl.ds(base, 8), :] = cy
            oz_ref[pl.ds(base, 8), :] = cz

        return dists, cidx, cx, cy, cz

    lax.fori_loop(0, S, body, (dists0, czero_i, czero_f, czero_f, czero_f))


def _run_fps(pos, interpret=False):
    pad = FPS_ROWS * 128 - N
    big = jnp.float32(1e9)
    coords = []
    for c in range(3):
        col = jnp.concatenate([pos[:, c], jnp.full((pad,), big, jnp.float32)])
        coords.append(col.reshape(FPS_ROWS, 128))
    out_shapes = (
        jax.ShapeDtypeStruct((OUT_ROWS, 128), jnp.int32),
        jax.ShapeDtypeStruct((OUT_ROWS, 128), jnp.float32),
        jax.ShapeDtypeStruct((OUT_ROWS, 128), jnp.float32),
        jax.ShapeDtypeStruct((OUT_ROWS, 128), jnp.float32),
    )
    idx_o, ox, oy, oz = pl.pallas_call(
        _fps_body,
        out_shape=out_shapes,
        interpret=interpret,
    )(*coords)
    fps_idx = idx_o.reshape(-1)[:S]
    pos_sub = jnp.stack(
        [ox.reshape(-1)[:S], oy.reshape(-1)[:S], oz.reshape(-1)[:S]], axis=1)
    return fps_idx, pos_sub


SP = 2528
QB = 32
PAIRS = SP * K


def _precomp_body(xp_ref, w_ref, g_ref):
    g_ref[:] = jnp.dot(xp_ref[:], w_ref[:],
                       preferred_element_type=jnp.float32,
                       precision=lax.Precision.HIGHEST)


def _run_precomp(x, pos, W1, interpret=False):
    xp = jnp.concatenate(
        [x, pos, jnp.zeros((N, 5), jnp.float32)], axis=1)
    w = jnp.concatenate([W1, jnp.zeros((5, 128), jnp.float32)], axis=0)
    return pl.pallas_call(
        _precomp_body,
        out_shape=jax.ShapeDtypeStruct((N, 128), jnp.float32),
        interpret=interpret,
    )(xp, w)


def _mlp_body(gg_ref, ps_ref, w1p_ref, b1_ref, w2_ref, b2_ref, out_ref):
    t = jnp.dot(ps_ref[:], w1p_ref[:],
                preferred_element_type=jnp.float32,
                precision=lax.Precision.HIGHEST)
    g3 = gg_ref[:].reshape(QB, K, 128)
    h1 = jnp.maximum(g3 - t[:, None, :] + b1_ref[:].reshape(1, 1, 128), 0.0)
    h2 = jnp.dot(h1.reshape(QB * K, 128), w2_ref[:],
                 preferred_element_type=jnp.float32,
                 precision=lax.Precision.HIGHEST) + b2_ref[:]
    out_ref[:] = jnp.max(h2.reshape(QB, K, 128), axis=1)


def _run_mlp(gg, ps_pad, W1, b1, W2, b2, interpret=False):
    w1p = jnp.concatenate(
        [W1[D:D + 3], jnp.zeros((5, 128), jnp.float32)], axis=0)
    nblk = SP // QB
    return pl.pallas_call(
        _mlp_body,
        grid=(nblk,),
        in_specs=[
            pl.BlockSpec((QB * K, 128), lambda i: (i, 0)),
            pl.BlockSpec((QB, 8), lambda i: (i, 0)),
            pl.BlockSpec((8, 128), lambda i: (0, 0)),
            pl.BlockSpec((1, 128), lambda i: (0, 0)),
            pl.BlockSpec((128, 128), lambda i: (0, 0)),
            pl.BlockSpec((1, 128), lambda i: (0, 0)),
        ],
        out_specs=pl.BlockSpec((QB, 128), lambda i: (i, 0)),
        out_shape=jax.ShapeDtypeStruct((SP, 128), jnp.float32),
        interpret=interpret,
    )(gg, ps_pad, w1p, b1.reshape(1, 128), W2, b2.reshape(1, 128))


def kernel(x, pos, batch, W1, b1, W2, b2):
    fps_idx, pos_sub = _run_fps(pos)
    g = _run_precomp(x, pos, W1)
    d2 = jnp.sum((pos_sub[:, None, :] - pos[None, :, :]) ** 2, axis=-1)
    neg = jnp.where(d2 <= RADIUS * RADIUS, -d2, -jnp.inf)
    vals, nbr = jax.lax.top_k(neg, K)
    nbr = jnp.where(vals > -jnp.inf, nbr, fps_idx[:, None])
    nbr_pad = jnp.pad(nbr, ((0, SP - S), (0, 0)))
    gg = g[nbr_pad.reshape(-1)]
    ps_pad = jnp.pad(pos_sub, ((0, SP - S), (0, 5)))
    out = _run_mlp(gg, ps_pad, W1, b1, W2, b2)[:S]
    return (out, pos_sub, batch[fps_idx])


# speedup vs baseline: 12.4221x; 2.8259x over previous
import functools

import jax
import jax.numpy as jnp
from jax import lax
from jax.experimental import pallas as pl
from jax.experimental.pallas import tpu as pltpu
from jax.experimental.pallas import tpu_sc as plsc

N = 10000
S = 2500
D = 128
K = 32
RADIUS = 0.25

FPS_ROWS = 80
OUT_ROWS = 24
CHUNK = 1024


def _fps_body(px_ref, py_ref, pz_ref, idx_ref, ox_ref, oy_ref, oz_ref):
    px = px_ref[:]
    py = py_ref[:]
    pz = pz_ref[:]
    shape = (FPS_ROWS, 128)
    lin = (lax.broadcasted_iota(jnp.int32, shape, 0) * 128
           + lax.broadcasted_iota(jnp.int32, shape, 1))
    valid = lin < N
    dists0 = jnp.where(valid, jnp.inf, -1.0).astype(jnp.float32)

    cshape = (8, 128)
    clin = (lax.broadcasted_iota(jnp.int32, cshape, 0) * 128
            + lax.broadcasted_iota(jnp.int32, cshape, 1))
    czero_i = jnp.zeros(cshape, jnp.int32)
    czero_f = jnp.zeros(cshape, jnp.float32)

    def body(i, carry):
        dists, cidx, cx, cy, cz = carry
        m = jnp.max(dists)
        sel = dists == m
        nxt = jnp.min(jnp.where(sel, lin, jnp.int32(2**30)))
        one = lin == nxt
        qx = jnp.sum(jnp.where(one, px, 0.0))
        qy = jnp.sum(jnp.where(one, py, 0.0))
        qz = jnp.sum(jnp.where(one, pz, 0.0))
        dx = px - qx
        dy = py - qy
        dz = pz - qz
        dnew = (dx * dx + dy * dy) + dz * dz
        dists = jnp.minimum(dists, dnew)
        slot = lax.rem(i, CHUNK)
        hit = clin == slot
        cidx = jnp.where(hit, nxt, cidx)
        cx = jnp.where(hit, qx, cx)
        cy = jnp.where(hit, qy, cy)
        cz = jnp.where(hit, qz, cz)

        @pl.when((slot == CHUNK - 1) | (i == S - 1))
        def _flush():
            base = lax.div(i, CHUNK) * 8
            idx_ref[pl.ds(base, 8), :] = cidx
            ox_ref[pl.ds(base, 8), :] = cx
            oy_ref[pl.ds(base, 8), :] = cy
            oz_ref[pl.ds(base, 8), :] = cz

        return dists, cidx, cx, cy, cz

    lax.fori_loop(0, S, body, (dists0, czero_i, czero_f, czero_f, czero_f))


def _run_fps(pos, interpret=False):
    pad = FPS_ROWS * 128 - N
    big = jnp.float32(1e9)
    coords = []
    for c in range(3):
        col = jnp.concatenate([pos[:, c], jnp.full((pad,), big, jnp.float32)])
        coords.append(col.reshape(FPS_ROWS, 128))
    out_shapes = (
        jax.ShapeDtypeStruct((OUT_ROWS, 128), jnp.int32),
        jax.ShapeDtypeStruct((OUT_ROWS, 128), jnp.float32),
        jax.ShapeDtypeStruct((OUT_ROWS, 128), jnp.float32),
        jax.ShapeDtypeStruct((OUT_ROWS, 128), jnp.float32),
    )
    idx_o, ox, oy, oz = pl.pallas_call(
        _fps_body,
        out_shape=out_shapes,
        interpret=interpret,
    )(*coords)
    fps_idx = idx_o.reshape(-1)[:S]
    pos_sub = jnp.stack(
        [ox.reshape(-1)[:S], oy.reshape(-1)[:S], oz.reshape(-1)[:S]], axis=1)
    return fps_idx, pos_sub


SP = 2560
QB = 32
NW = 32
QW = SP // NW
LANES = 16
CAP = N // LANES
R2 = RADIUS * RADIUS


def _precomp_body(xp_ref, w_ref, g_ref):
    g_ref[:] = jnp.dot(xp_ref[:], w_ref[:],
                       preferred_element_type=jnp.float32,
                       precision=lax.Precision.HIGHEST)


def _run_precomp(x, pos, W1, interpret=False):
    xp = jnp.concatenate(
        [x, pos, jnp.zeros((N, 5), jnp.float32)], axis=1)
    w = jnp.concatenate([W1, jnp.zeros((5, 128), jnp.float32)], axis=0)
    return pl.pallas_call(
        _precomp_body,
        out_shape=jax.ShapeDtypeStruct((N, 128), jnp.float32),
        interpret=interpret,
    )(xp, w)


def _nbr_body(px_hbm, py_hbm, pz_hbm, psx_hbm, psy_hbm, psz_hbm,
              nbr_hbm, pxv, pyv, pzv, qxv, qyv, qzv, cd, cj, stage):
    wid = lax.axis_index("s") * 2 + lax.axis_index("c")
    pltpu.sync_copy(px_hbm, pxv)
    pltpu.sync_copy(py_hbm, pyv)
    pltpu.sync_copy(pz_hbm, pzv)
    pltpu.sync_copy(psx_hbm.at[pl.ds(wid * QW, QW)], qxv)
    pltpu.sync_copy(psy_hbm.at[pl.ds(wid * QW, QW)], qyv)
    pltpu.sync_copy(psz_hbm.at[pl.ds(wid * QW, QW)], qzv)
    lane = lax.iota(jnp.int32, LANES)
    lane0 = lane == 0
    INF = jnp.float32(jnp.inf)
    BIGI = jnp.int32(2**30)

    def qbody(q, _):
        qb = (q // LANES) * LANES
        qsel = lane == q - qb
        qx = jnp.full((LANES,),
                      jnp.sum(jnp.where(qsel, qxv[pl.ds(qb, LANES)], 0.0)))
        qy = jnp.full((LANES,),
                      jnp.sum(jnp.where(qsel, qyv[pl.ds(qb, LANES)], 0.0)))
        qz = jnp.full((LANES,),
                      jnp.sum(jnp.where(qsel, qzv[pl.ds(qb, LANES)], 0.0)))

        def scan_block(b, lcnt):
            base = b * LANES
            dx = pxv[pl.ds(base, LANES)] - qx
            dy = pyv[pl.ds(base, LANES)] - qy
            dz = pzv[pl.ds(base, LANES)] - qz
            d2 = (dx * dx + dy * dy) + dz * dz
            msk = d2 <= R2
            addr = lcnt * LANES + lane
            plsc.store_scatter(cd, [addr], d2, mask=msk)
            plsc.store_scatter(cj, [addr], base + lane, mask=msk)
            return lcnt + msk.astype(jnp.int32)

        lcnt = lax.fori_loop(0, N // LANES, scan_block,
                             jnp.zeros((LANES,), jnp.int32))
        maxc = jnp.max(lcnt)

        def ext_body(k, fill):
            def row_body(cc, st):
                bd, bj, ba = st
                base = cc * LANES
                d = jnp.where(cc < lcnt, cd[pl.ds(base, LANES)], INF)
                jr = cj[pl.ds(base, LANES)]
                better = (d < bd) | ((d == bd) & (jr < bj))
                return (jnp.where(better, d, bd),
                        jnp.where(better, jr, bj),
                        jnp.where(better, base + lane, ba))

            bd, bj, ba = lax.fori_loop(
                0, maxc, row_body,
                (jnp.full((LANES,), INF),
                 jnp.full((LANES,), BIGI),
                 jnp.zeros((LANES,), jnp.int32)))
            m = jnp.min(bd)
            elig = bd == m
            jm = jnp.min(jnp.where(elig, bj, BIGI))
            am = jnp.min(jnp.where(elig & (bj == jm), ba, BIGI))
            found = m < INF
            am_s = jnp.where(found, am, 0)
            plsc.store_scatter(cd, [jnp.full((LANES,), am_s, jnp.int32)],
                               jnp.full((LANES,), INF), mask=lane0)
            fill = jnp.where((k == 0) & found, jm, fill)
            jout = jnp.where(found, jm, fill)
            plsc.store_scatter(stage,
                               [jnp.full((LANES,), q * K + k, jnp.int32)],
                               jnp.full((LANES,), jout, jnp.int32),
                               mask=lane0)
            return fill

        lax.fori_loop(0, K, ext_body, jnp.int32(0))
        return 0

    lax.fori_loop(0, QW, qbody, 0)
    pltpu.sync_copy(stage, nbr_hbm.at[pl.ds(wid * QW * K, QW * K)])


def _run_nbr(px, py, pz, psx, psy, psz):
    mesh = plsc.VectorSubcoreMesh(core_axis_name="c", subcore_axis_name="s")
    f32, i32 = jnp.float32, jnp.int32
    kfn = functools.partial(
        pl.kernel, mesh=mesh,
        compiler_params=pltpu.CompilerParams(needs_layout_passes=False),
        out_type=jax.ShapeDtypeStruct((SP * K,), i32),
        scratch_types=[
            pltpu.VMEM((N,), f32), pltpu.VMEM((N,), f32),
            pltpu.VMEM((N,), f32),
            pltpu.VMEM((QW,), f32), pltpu.VMEM((QW,), f32),
            pltpu.VMEM((QW,), f32),
            pltpu.VMEM((CAP * LANES,), f32),
            pltpu.VMEM((CAP * LANES,), i32),
            pltpu.VMEM((QW * K,), i32),
        ],
    )(_nbr_body)
    return kfn(px, py, pz, psx, psy, psz)


GCH = 512


def _gather_body(g_hbm, nbr_hbm, fidx_hbm, batch_hbm, gg_hbm, bsub_hbm,
                 idxv, rows, bvec, fvec, bout, sem):
    wid = lax.axis_index("s") * 2 + lax.axis_index("c")
    base = wid * QW * K
    pltpu.sync_copy(nbr_hbm.at[pl.ds(base, QW * K)], idxv)

    def chunk(i, _):
        co = i * GCH
        pltpu.async_copy(g_hbm.at[idxv.at[pl.ds(co, GCH)]], rows, sem).wait()
        pltpu.sync_copy(rows, gg_hbm.at[pl.ds(base + co, GCH)])
        return 0

    lax.fori_loop(0, QW * K // GCH, chunk, 0)

    @pl.when(wid == 0)
    def _batch():
        pltpu.sync_copy(batch_hbm, bvec)
        pltpu.sync_copy(fidx_hbm, fvec)

        def bb(b, _):
            iv = fvec[pl.ds(b * LANES, LANES)]
            bout[pl.ds(b * LANES, LANES)] = plsc.load_gather(bvec, [iv])
            return 0

        lax.fori_loop(0, SP // LANES, bb, 0)
        pltpu.sync_copy(bout, bsub_hbm)


def _run_gather(g, nbr_flat, fidx_pad, batch):
    mesh = plsc.VectorSubcoreMesh(core_axis_name="c", subcore_axis_name="s")
    f32, i32 = jnp.float32, jnp.int32
    kfn = functools.partial(
        pl.kernel, mesh=mesh,
        compiler_params=pltpu.CompilerParams(needs_layout_passes=False),
        out_type=(jax.ShapeDtypeStruct((SP * K, 128), f32),
                  jax.ShapeDtypeStruct((SP,), i32)),
        scratch_types=[
            pltpu.VMEM((QW * K,), i32),
            pltpu.VMEM((GCH, 128), f32),
            pltpu.VMEM((N,), i32),
            pltpu.VMEM((SP,), i32),
            pltpu.VMEM((SP,), i32),
            pltpu.SemaphoreType.DMA,
        ],
    )(_gather_body)
    return kfn(g, nbr_flat, fidx_pad, batch)


def _mlp_body(gg_ref, ps_ref, w1p_ref, b1_ref, w2_ref, b2_ref, out_ref):
    t = jnp.dot(ps_ref[:], w1p_ref[:],
                preferred_element_type=jnp.float32,
                precision=lax.Precision.HIGHEST)
    g3 = gg_ref[:].reshape(QB, K, 128)
    h1 = jnp.maximum(g3 - t[:, None, :] + b1_ref[:].reshape(1, 1, 128), 0.0)
    h2 = jnp.dot(h1.reshape(QB * K, 128), w2_ref[:],
                 preferred_element_type=jnp.float32,
                 precision=lax.Precision.HIGHEST) + b2_ref[:]
    out_ref[:] = jnp.max(h2.reshape(QB, K, 128), axis=1)


def _run_mlp(gg, ps_pad, W1, b1, W2, b2, interpret=False):
    w1p = jnp.concatenate(
        [W1[D:D + 3], jnp.zeros((5, 128), jnp.float32)], axis=0)
    nblk = SP // QB
    return pl.pallas_call(
        _mlp_body,
        grid=(nblk,),
        in_specs=[
            pl.BlockSpec((QB * K, 128), lambda i: (i, 0)),
            pl.BlockSpec((QB, 8), lambda i: (i, 0)),
            pl.BlockSpec((8, 128), lambda i: (0, 0)),
            pl.BlockSpec((1, 128), lambda i: (0, 0)),
            pl.BlockSpec((128, 128), lambda i: (0, 0)),
            pl.BlockSpec((1, 128), lambda i: (0, 0)),
        ],
        out_specs=pl.BlockSpec((QB, 128), lambda i: (i, 0)),
        out_shape=jax.ShapeDtypeStruct((SP, 128), jnp.float32),
        interpret=interpret,
    )(gg, ps_pad, w1p, b1.reshape(1, 128), W2, b2.reshape(1, 128))


def kernel(x, pos, batch, W1, b1, W2, b2):
    fps_idx, pos_sub = _run_fps(pos)
    g = _run_precomp(x, pos, W1)
    big = jnp.float32(1e9)
    padq = jnp.full((SP - S,), big, jnp.float32)
    psx = jnp.concatenate([pos_sub[:, 0], padq])
    psy = jnp.concatenate([pos_sub[:, 1], padq])
    psz = jnp.concatenate([pos_sub[:, 2], padq])
    nbr_flat = _run_nbr(pos[:, 0], pos[:, 1], pos[:, 2], psx, psy, psz)
    fidx_pad = jnp.pad(fps_idx, (0, SP - S))
    gg, bsub = _run_gather(g, nbr_flat, fidx_pad, batch)
    ps_pad = jnp.pad(pos_sub, ((0, SP - S), (0, 5)))
    out = _run_mlp(gg, ps_pad, W1, b1, W2, b2)[:S]
    return (out, pos_sub, bsub[:S])


# speedup vs baseline: 12.5675x; 1.0117x over previous
import functools

import jax
import jax.numpy as jnp
from jax import lax
from jax.experimental import pallas as pl
from jax.experimental.pallas import tpu as pltpu
from jax.experimental.pallas import tpu_sc as plsc

N = 10000
S = 2500
D = 128
K = 32
RADIUS = 0.25

FPS_ROWS = 80
OUT_ROWS = 24
CHUNK = 1024


def _fps_body(px_ref, py_ref, pz_ref, idx_ref, ox_ref, oy_ref, oz_ref):
    px = px_ref[:]
    py = py_ref[:]
    pz = pz_ref[:]
    shape = (FPS_ROWS, 128)
    lin = (lax.broadcasted_iota(jnp.int32, shape, 0) * 128
           + lax.broadcasted_iota(jnp.int32, shape, 1))
    valid = lin < N
    dists0 = jnp.where(valid, jnp.inf, -1.0).astype(jnp.float32)

    cshape = (8, 128)
    clin = (lax.broadcasted_iota(jnp.int32, cshape, 0) * 128
            + lax.broadcasted_iota(jnp.int32, cshape, 1))
    czero_i = jnp.zeros(cshape, jnp.int32)
    czero_f = jnp.zeros(cshape, jnp.float32)

    def body(i, carry):
        dists, cidx, cx, cy, cz = carry
        m = jnp.max(dists)
        sel = dists == m
        nxt = jnp.min(jnp.where(sel, lin, jnp.int32(2**30)))
        one = lin == nxt
        qx = jnp.sum(jnp.where(one, px, 0.0))
        qy = jnp.sum(jnp.where(one, py, 0.0))
        qz = jnp.sum(jnp.where(one, pz, 0.0))
        dx = px - qx
        dy = py - qy
        dz = pz - qz
        dnew = (dx * dx + dy * dy) + dz * dz
        dists = jnp.minimum(dists, dnew)
        slot = lax.rem(i, CHUNK)
        hit = clin == slot
        cidx = jnp.where(hit, nxt, cidx)
        cx = jnp.where(hit, qx, cx)
        cy = jnp.where(hit, qy, cy)
        cz = jnp.where(hit, qz, cz)

        @pl.when((slot == CHUNK - 1) | (i == S - 1))
        def _flush():
            base = lax.div(i, CHUNK) * 8
            idx_ref[pl.ds(base, 8), :] = cidx
            ox_ref[pl.ds(base, 8), :] = cx
            oy_ref[pl.ds(base, 8), :] = cy
            oz_ref[pl.ds(base, 8), :] = cz

        return dists, cidx, cx, cy, cz

    lax.fori_loop(0, S, body, (dists0, czero_i, czero_f, czero_f, czero_f))


def _run_fps(pos, interpret=False):
    pad = FPS_ROWS * 128 - N
    big = jnp.float32(1e9)
    coords = []
    for c in range(3):
        col = jnp.concatenate([pos[:, c], jnp.full((pad,), big, jnp.float32)])
        coords.append(col.reshape(FPS_ROWS, 128))
    out_shapes = (
        jax.ShapeDtypeStruct((OUT_ROWS, 128), jnp.int32),
        jax.ShapeDtypeStruct((OUT_ROWS, 128), jnp.float32),
        jax.ShapeDtypeStruct((OUT_ROWS, 128), jnp.float32),
        jax.ShapeDtypeStruct((OUT_ROWS, 128), jnp.float32),
    )
    idx_o, ox, oy, oz = pl.pallas_call(
        _fps_body,
        out_shape=out_shapes,
        interpret=interpret,
    )(*coords)
    fps_idx = idx_o.reshape(-1)[:S]
    pos_sub = jnp.stack(
        [ox.reshape(-1)[:S], oy.reshape(-1)[:S], oz.reshape(-1)[:S]], axis=1)
    return fps_idx, pos_sub


SP = 2560
QB = 32
NW = 32
QW = SP // NW
LANES = 16
CAP = N // LANES
R2 = RADIUS * RADIUS


def _precomp_body(xp_ref, w_ref, g_ref):
    g_ref[:] = jnp.dot(xp_ref[:], w_ref[:],
                       preferred_element_type=jnp.float32,
                       precision=lax.Precision.HIGHEST)


def _run_precomp(x, pos, W1, interpret=False):
    xp = jnp.concatenate(
        [x, pos, jnp.zeros((N, 5), jnp.float32)], axis=1)
    w = jnp.concatenate([W1, jnp.zeros((5, 128), jnp.float32)], axis=0)
    return pl.pallas_call(
        _precomp_body,
        out_shape=jax.ShapeDtypeStruct((N, 128), jnp.float32),
        interpret=interpret,
    )(xp, w)


NP = 10112
BIGJ = jnp.int32(N)
UNROLL = 25


def _nbr_body(px_hbm, py_hbm, pz_hbm, psx_hbm, psy_hbm, psz_hbm,
              nbr_hbm, pxv, pyv, pzv, qxv, qyv, qzv, cj, stage):
    wid = lax.axis_index("s") * 2 + lax.axis_index("c")
    pltpu.sync_copy(px_hbm, pxv)
    pltpu.sync_copy(py_hbm, pyv)
    pltpu.sync_copy(pz_hbm, pzv)
    pltpu.sync_copy(psx_hbm.at[pl.ds(wid * QW, QW)], qxv.at[pl.ds(0, QW)])
    pltpu.sync_copy(psy_hbm.at[pl.ds(wid * QW, QW)], qyv.at[pl.ds(0, QW)])
    pltpu.sync_copy(psz_hbm.at[pl.ds(wid * QW, QW)], qzv.at[pl.ds(0, QW)])
    lane = lax.iota(jnp.int32, LANES)
    lane0 = lane == 0
    INF = jnp.float32(jnp.inf)
    BIGI = jnp.int32(2**30)

    def qbody(q, _):
        qb = (q // LANES) * LANES
        qsel = lane == q - qb
        qx = jnp.full((LANES,),
                      jnp.sum(jnp.where(qsel, qxv[pl.ds(qb, LANES)], 0.0)))
        qy = jnp.full((LANES,),
                      jnp.sum(jnp.where(qsel, qyv[pl.ds(qb, LANES)], 0.0)))
        qz = jnp.full((LANES,),
                      jnp.sum(jnp.where(qsel, qzv[pl.ds(qb, LANES)], 0.0)))

        def scan_chunk(bo, lcnt):
            for u in range(UNROLL):
                base = (bo * UNROLL + u) * LANES
                dx = pxv[pl.ds(base, LANES)] - qx
                dy = pyv[pl.ds(base, LANES)] - qy
                dz = pzv[pl.ds(base, LANES)] - qz
                d2 = (dx * dx + dy * dy) + dz * dz
                msk = d2 <= R2
                addr = lcnt * LANES + lane
                plsc.store_scatter(cj, [addr], base + lane, mask=msk)
                lcnt = lcnt + msk.astype(jnp.int32)
            return lcnt

        lcnt = lax.fori_loop(0, N // LANES // UNROLL, scan_chunk,
                             jnp.zeros((LANES,), jnp.int32))
        maxc = jnp.max(lcnt)

        def ext_body(k, fill):
            def row_body(cc, st):
                bd, bj, ba = st
                base = cc * LANES
                jr = cj[pl.ds(base, LANES)]
                js = jnp.minimum(jnp.maximum(jr, 0), jnp.int32(NP - 1))
                dxj = plsc.load_gather(pxv, [js]) - qx
                dyj = plsc.load_gather(pyv, [js]) - qy
                dzj = plsc.load_gather(pzv, [js]) - qz
                d2j = (dxj * dxj + dyj * dyj) + dzj * dzj
                d = jnp.where(cc < lcnt, d2j, INF)
                better = (d < bd) | ((d == bd) & (jr < bj))
                return (jnp.where(better, d, bd),
                        jnp.where(better, jr, bj),
                        jnp.where(better, base + lane, ba))

            bd, bj, ba = lax.fori_loop(
                0, maxc, row_body,
                (jnp.full((LANES,), INF),
                 jnp.full((LANES,), BIGI),
                 jnp.zeros((LANES,), jnp.int32)))
            m = jnp.min(bd)
            elig = bd == m
            jm = jnp.min(jnp.where(elig, bj, BIGI))
            am = jnp.min(jnp.where(elig & (bj == jm), ba, BIGI))
            found = m <= R2
            am_s = jnp.where(found, am, 0)
            plsc.store_scatter(cj, [jnp.full((LANES,), am_s, jnp.int32)],
                               jnp.full((LANES,), BIGJ), mask=lane0)
            fill = jnp.where((k == 0) & found, jm, fill)
            jout = jnp.where(found, jm, fill)
            plsc.store_scatter(stage,
                               [jnp.full((LANES,), q * K + k, jnp.int32)],
                               jnp.full((LANES,), jout, jnp.int32),
                               mask=lane0)
            return fill

        lax.fori_loop(0, K, ext_body, jnp.int32(0))
        return 0

    lax.fori_loop(0, QW, qbody, 0)
    pltpu.sync_copy(stage, nbr_hbm.at[pl.ds(wid * QW * K, QW * K)])


def _run_nbr(px, py, pz, psx, psy, psz):
    mesh = plsc.VectorSubcoreMesh(core_axis_name="c", subcore_axis_name="s")
    f32, i32 = jnp.float32, jnp.int32
    kfn = functools.partial(
        pl.kernel, mesh=mesh,
        compiler_params=pltpu.CompilerParams(needs_layout_passes=False),
        out_type=jax.ShapeDtypeStruct((SP * K,), i32),
        scratch_types=[
            pltpu.VMEM((NP,), f32), pltpu.VMEM((NP,), f32),
            pltpu.VMEM((NP,), f32),
            pltpu.VMEM((128,), f32), pltpu.VMEM((128,), f32),
            pltpu.VMEM((128,), f32),
            pltpu.VMEM((CAP * LANES + 128,), i32),
            pltpu.VMEM((QW * K,), i32),
        ],
    )(_nbr_body)
    return kfn(px, py, pz, psx, psy, psz)


GCH = 512


def _gather_body(g_hbm, nbr_hbm, fidx_hbm, batch_hbm, gg_hbm, bsub_hbm,
                 idxv, rows, bvec, fvec, bout, sem):
    wid = lax.axis_index("s") * 2 + lax.axis_index("c")
    base = wid * QW * K
    pltpu.sync_copy(nbr_hbm.at[pl.ds(base, QW * K)], idxv)

    def chunk(i, _):
        co = i * GCH
        pltpu.async_copy(g_hbm.at[idxv.at[pl.ds(co, GCH)]], rows, sem).wait()
        pltpu.sync_copy(rows, gg_hbm.at[pl.ds(base + co, GCH)])
        return 0

    lax.fori_loop(0, QW * K // GCH, chunk, 0)

    @pl.when(wid == 0)
    def _batch():
        pltpu.sync_copy(batch_hbm, bvec)
        pltpu.sync_copy(fidx_hbm, fvec)

        def bb(b, _):
            iv = fvec[pl.ds(b * LANES, LANES)]
            bout[pl.ds(b * LANES, LANES)] = plsc.load_gather(bvec, [iv])
            return 0

        lax.fori_loop(0, SP // LANES, bb, 0)
        pltpu.sync_copy(bout, bsub_hbm)


def _run_gather(g, nbr_flat, fidx_pad, batch):
    mesh = plsc.VectorSubcoreMesh(core_axis_name="c", subcore_axis_name="s")
    f32, i32 = jnp.float32, jnp.int32
    kfn = functools.partial(
        pl.kernel, mesh=mesh,
        compiler_params=pltpu.CompilerParams(needs_layout_passes=False),
        out_type=(jax.ShapeDtypeStruct((SP * K, 128), f32),
                  jax.ShapeDtypeStruct((SP,), i32)),
        scratch_types=[
            pltpu.VMEM((QW * K,), i32),
            pltpu.VMEM((GCH, 128), f32),
            pltpu.VMEM((N,), i32),
            pltpu.VMEM((SP,), i32),
            pltpu.VMEM((SP,), i32),
            pltpu.SemaphoreType.DMA,
        ],
    )(_gather_body)
    return kfn(g, nbr_flat, fidx_pad, batch)


def _mlp_body(gg_ref, ps_ref, w1p_ref, b1_ref, w2_ref, b2_ref, out_ref):
    t = jnp.dot(ps_ref[:], w1p_ref[:],
                preferred_element_type=jnp.float32,
                precision=lax.Precision.HIGHEST)
    g3 = gg_ref[:].reshape(QB, K, 128)
    h1 = jnp.maximum(g3 - t[:, None, :] + b1_ref[:].reshape(1, 1, 128), 0.0)
    h2 = jnp.dot(h1.reshape(QB * K, 128), w2_ref[:],
                 preferred_element_type=jnp.float32,
                 precision=lax.Precision.HIGHEST) + b2_ref[:]
    out_ref[:] = jnp.max(h2.reshape(QB, K, 128), axis=1)


def _run_mlp(gg, ps_pad, W1, b1, W2, b2, interpret=False):
    w1p = jnp.concatenate(
        [W1[D:D + 3], jnp.zeros((5, 128), jnp.float32)], axis=0)
    nblk = SP // QB
    return pl.pallas_call(
        _mlp_body,
        grid=(nblk,),
        in_specs=[
            pl.BlockSpec((QB * K, 128), lambda i: (i, 0)),
            pl.BlockSpec((QB, 8), lambda i: (i, 0)),
            pl.BlockSpec((8, 128), lambda i: (0, 0)),
            pl.BlockSpec((1, 128), lambda i: (0, 0)),
            pl.BlockSpec((128, 128), lambda i: (0, 0)),
            pl.BlockSpec((1, 128), lambda i: (0, 0)),
        ],
        out_specs=pl.BlockSpec((QB, 128), lambda i: (i, 0)),
        out_shape=jax.ShapeDtypeStruct((SP, 128), jnp.float32),
        interpret=interpret,
    )(gg, ps_pad, w1p, b1.reshape(1, 128), W2, b2.reshape(1, 128))


def kernel(x, pos, batch, W1, b1, W2, b2):
    fps_idx, pos_sub = _run_fps(pos)
    g = _run_precomp(x, pos, W1)
    big = jnp.float32(1e9)
    padq = jnp.full((SP - S,), big, jnp.float32)
    psx = jnp.concatenate([pos_sub[:, 0], padq])
    psy = jnp.concatenate([pos_sub[:, 1], padq])
    psz = jnp.concatenate([pos_sub[:, 2], padq])
    padp = jnp.full((NP - N,), big, jnp.float32)
    nbr_flat = _run_nbr(jnp.concatenate([pos[:, 0], padp]),
                        jnp.concatenate([pos[:, 1], padp]),
                        jnp.concatenate([pos[:, 2], padp]),
                        psx, psy, psz)
    fidx_pad = jnp.pad(fps_idx, (0, SP - S))
    gg, bsub = _run_gather(g, nbr_flat, fidx_pad, batch)
    ps_pad = jnp.pad(pos_sub, ((0, SP - S), (0, 5)))
    out = _run_mlp(gg, ps_pad, W1, b1, W2, b2)[:S]
    return (out, pos_sub, bsub[:S])


# speedup vs baseline: 15.8976x; 1.2650x over previous
import functools

import jax
import jax.numpy as jnp
from jax import lax
from jax.experimental import pallas as pl
from jax.experimental.pallas import tpu as pltpu
from jax.experimental.pallas import tpu_sc as plsc

N = 10000
S = 2500
D = 128
K = 32
RADIUS = 0.25

FPS_ROWS = 80
OUT_ROWS = 24
CHUNK = 1024


def _fps_body(px_ref, py_ref, pz_ref, idx_ref, ox_ref, oy_ref, oz_ref):
    px = px_ref[:]
    py = py_ref[:]
    pz = pz_ref[:]
    shape = (FPS_ROWS, 128)
    lin = (lax.broadcasted_iota(jnp.int32, shape, 0) * 128
           + lax.broadcasted_iota(jnp.int32, shape, 1))
    valid = lin < N
    dists0 = jnp.where(valid, jnp.inf, -1.0).astype(jnp.float32)

    cshape = (8, 128)
    clin = (lax.broadcasted_iota(jnp.int32, cshape, 0) * 128
            + lax.broadcasted_iota(jnp.int32, cshape, 1))
    czero_i = jnp.zeros(cshape, jnp.int32)
    czero_f = jnp.zeros(cshape, jnp.float32)

    def body(i, carry):
        dists, cidx, cx, cy, cz = carry
        m = jnp.max(dists)
        sel = dists == m
        nxt = jnp.min(jnp.where(sel, lin, jnp.int32(2**30)))
        one = lin == nxt
        qx = jnp.sum(jnp.where(one, px, 0.0))
        qy = jnp.sum(jnp.where(one, py, 0.0))
        qz = jnp.sum(jnp.where(one, pz, 0.0))
        dx = px - qx
        dy = py - qy
        dz = pz - qz
        dnew = (dx * dx + dy * dy) + dz * dz
        dists = jnp.minimum(dists, dnew)
        slot = lax.rem(i, CHUNK)
        hit = clin == slot
        cidx = jnp.where(hit, nxt, cidx)
        cx = jnp.where(hit, qx, cx)
        cy = jnp.where(hit, qy, cy)
        cz = jnp.where(hit, qz, cz)

        @pl.when((slot == CHUNK - 1) | (i == S - 1))
        def _flush():
            base = lax.div(i, CHUNK) * 8
            idx_ref[pl.ds(base, 8), :] = cidx
            ox_ref[pl.ds(base, 8), :] = cx
            oy_ref[pl.ds(base, 8), :] = cy
            oz_ref[pl.ds(base, 8), :] = cz

        return dists, cidx, cx, cy, cz

    lax.fori_loop(0, S, body, (dists0, czero_i, czero_f, czero_f, czero_f))


def _run_fps(pos, interpret=False):
    pad = FPS_ROWS * 128 - N
    big = jnp.float32(1e9)
    coords = []
    for c in range(3):
        col = jnp.concatenate([pos[:, c], jnp.full((pad,), big, jnp.float32)])
        coords.append(col.reshape(FPS_ROWS, 128))
    out_shapes = (
        jax.ShapeDtypeStruct((OUT_ROWS, 128), jnp.int32),
        jax.ShapeDtypeStruct((OUT_ROWS, 128), jnp.float32),
        jax.ShapeDtypeStruct((OUT_ROWS, 128), jnp.float32),
        jax.ShapeDtypeStruct((OUT_ROWS, 128), jnp.float32),
    )
    idx_o, ox, oy, oz = pl.pallas_call(
        _fps_body,
        out_shape=out_shapes,
        interpret=interpret,
    )(*coords)
    fps_idx = idx_o.reshape(-1)[:S]
    pos_sub = jnp.stack(
        [ox.reshape(-1)[:S], oy.reshape(-1)[:S], oz.reshape(-1)[:S]], axis=1)
    return fps_idx, pos_sub


SP = 2560
QB = 32
NW = 32
QW = SP // NW
LANES = 16
CAP = N // LANES
R2 = RADIUS * RADIUS


def _precomp_body(xp_ref, w_ref, g_ref):
    g_ref[:] = jnp.dot(xp_ref[:], w_ref[:],
                       preferred_element_type=jnp.float32,
                       precision=lax.Precision.HIGHEST)


def _run_precomp(x, pos, W1, interpret=False):
    xp = jnp.concatenate(
        [x, pos, jnp.zeros((N, 5), jnp.float32)], axis=1)
    w = jnp.concatenate([W1, jnp.zeros((5, 128), jnp.float32)], axis=0)
    return pl.pallas_call(
        _precomp_body,
        out_shape=jax.ShapeDtypeStruct((N, 128), jnp.float32),
        interpret=interpret,
    )(xp, w)


NP = 10112
BIGJ = jnp.int32(N)
UNROLL = 8


def _nbr_body(px_hbm, py_hbm, pz_hbm, psx_hbm, psy_hbm, psz_hbm,
              nbr_hbm, pxv, pyv, pzv, qxv, qyv, qzv, cj, stage):
    wid = lax.axis_index("s") * 2 + lax.axis_index("c")
    pltpu.sync_copy(px_hbm, pxv)
    pltpu.sync_copy(py_hbm, pyv)
    pltpu.sync_copy(pz_hbm, pzv)
    pltpu.sync_copy(psx_hbm.at[pl.ds(wid * QW, QW)], qxv.at[pl.ds(0, QW)])
    pltpu.sync_copy(psy_hbm.at[pl.ds(wid * QW, QW)], qyv.at[pl.ds(0, QW)])
    pltpu.sync_copy(psz_hbm.at[pl.ds(wid * QW, QW)], qzv.at[pl.ds(0, QW)])
    lane = lax.iota(jnp.int32, LANES)
    lane0 = lane == 0
    INF = jnp.float32(jnp.inf)
    BIGI = jnp.int32(2**30)

    def qbody(q, _):
        qb = (q // LANES) * LANES
        qsel = lane == q - qb
        qx = jnp.full((LANES,),
                      jnp.sum(jnp.where(qsel, qxv[pl.ds(qb, LANES)], 0.0)))
        qy = jnp.full((LANES,),
                      jnp.sum(jnp.where(qsel, qyv[pl.ds(qb, LANES)], 0.0)))
        qz = jnp.full((LANES,),
                      jnp.sum(jnp.where(qsel, qzv[pl.ds(qb, LANES)], 0.0)))

        @plsc.parallel_loop(0, N, step=LANES, unroll=UNROLL,
                            carry=jnp.zeros((LANES,), jnp.int32))
        def lcnt(base, lc):
            dx = pxv[pl.ds(base, LANES)] - qx
            dy = pyv[pl.ds(base, LANES)] - qy
            dz = pzv[pl.ds(base, LANES)] - qz
            d2 = (dx * dx + dy * dy) + dz * dz
            msk = d2 <= R2
            addr = lc * LANES + lane
            plsc.store_scatter(cj, [addr], base + lane, mask=msk)
            return lc + msk.astype(jnp.int32)

        maxc = jnp.max(lcnt)

        def ext_body(k, fill):
            def row_body(cc, st):
                bd, bj, ba = st
                base = cc * LANES
                jr = cj[pl.ds(base, LANES)]
                js = jnp.minimum(jnp.maximum(jr, 0), jnp.int32(NP - 1))
                dxj = plsc.load_gather(pxv, [js]) - qx
                dyj = plsc.load_gather(pyv, [js]) - qy
                dzj = plsc.load_gather(pzv, [js]) - qz
                d2j = (dxj * dxj + dyj * dyj) + dzj * dzj
                d = jnp.where(cc < lcnt, d2j, INF)
                better = (d < bd) | ((d == bd) & (jr < bj))
                return (jnp.where(better, d, bd),
                        jnp.where(better, jr, bj),
                        jnp.where(better, base + lane, ba))

            bd, bj, ba = lax.fori_loop(
                0, maxc, row_body,
                (jnp.full((LANES,), INF),
                 jnp.full((LANES,), BIGI),
                 jnp.zeros((LANES,), jnp.int32)))
            m = jnp.min(bd)
            elig = bd == m
            jm = jnp.min(jnp.where(elig, bj, BIGI))
            am = jnp.min(jnp.where(elig & (bj == jm), ba, BIGI))
            found = m <= R2
            am_s = jnp.where(found, am, 0)
            plsc.store_scatter(cj, [jnp.full((LANES,), am_s, jnp.int32)],
                               jnp.full((LANES,), BIGJ), mask=lane0)
            fill = jnp.where((k == 0) & found, jm, fill)
            jout = jnp.where(found, jm, fill)
            plsc.store_scatter(stage,
                               [jnp.full((LANES,), q * K + k, jnp.int32)],
                               jnp.full((LANES,), jout, jnp.int32),
                               mask=lane0)
            return fill

        lax.fori_loop(0, K, ext_body, jnp.int32(0))
        return 0

    lax.fori_loop(0, QW, qbody, 0)
    pltpu.sync_copy(stage, nbr_hbm.at[pl.ds(wid * QW * K, QW * K)])


def _run_nbr(px, py, pz, psx, psy, psz):
    mesh = plsc.VectorSubcoreMesh(core_axis_name="c", subcore_axis_name="s")
    f32, i32 = jnp.float32, jnp.int32
    kfn = functools.partial(
        pl.kernel, mesh=mesh,
        compiler_params=pltpu.CompilerParams(needs_layout_passes=False),
        out_type=jax.ShapeDtypeStruct((SP * K,), i32),
        scratch_types=[
            pltpu.VMEM((NP,), f32), pltpu.VMEM((NP,), f32),
            pltpu.VMEM((NP,), f32),
            pltpu.VMEM((128,), f32), pltpu.VMEM((128,), f32),
            pltpu.VMEM((128,), f32),
            pltpu.VMEM((CAP * LANES + 128,), i32),
            pltpu.VMEM((QW * K,), i32),
        ],
    )(_nbr_body)
    return kfn(px, py, pz, psx, psy, psz)


GCH = 512


def _gather_body(g_hbm, nbr_hbm, fidx_hbm, batch_hbm, gg_hbm, bsub_hbm,
                 idxv, rows, bvec, fvec, bout, sem):
    wid = lax.axis_index("s") * 2 + lax.axis_index("c")
    base = wid * QW * K
    pltpu.sync_copy(nbr_hbm.at[pl.ds(base, QW * K)], idxv)

    def chunk(i, _):
        co = i * GCH
        pltpu.async_copy(g_hbm.at[idxv.at[pl.ds(co, GCH)]], rows, sem).wait()
        pltpu.sync_copy(rows, gg_hbm.at[pl.ds(base + co, GCH)])
        return 0

    lax.fori_loop(0, QW * K // GCH, chunk, 0)

    @pl.when(wid == 0)
    def _batch():
        pltpu.sync_copy(batch_hbm, bvec)
        pltpu.sync_copy(fidx_hbm, fvec)

        def bb(b, _):
            iv = fvec[pl.ds(b * LANES, LANES)]
            bout[pl.ds(b * LANES, LANES)] = plsc.load_gather(bvec, [iv])
            return 0

        lax.fori_loop(0, SP // LANES, bb, 0)
        pltpu.sync_copy(bout, bsub_hbm)


def _run_gather(g, nbr_flat, fidx_pad, batch):
    mesh = plsc.VectorSubcoreMesh(core_axis_name="c", subcore_axis_name="s")
    f32, i32 = jnp.float32, jnp.int32
    kfn = functools.partial(
        pl.kernel, mesh=mesh,
        compiler_params=pltpu.CompilerParams(needs_layout_passes=False),
        out_type=(jax.ShapeDtypeStruct((SP * K, 128), f32),
                  jax.ShapeDtypeStruct((SP,), i32)),
        scratch_types=[
            pltpu.VMEM((QW * K,), i32),
            pltpu.VMEM((GCH, 128), f32),
            pltpu.VMEM((N,), i32),
            pltpu.VMEM((SP,), i32),
            pltpu.VMEM((SP,), i32),
            pltpu.SemaphoreType.DMA,
        ],
    )(_gather_body)
    return kfn(g, nbr_flat, fidx_pad, batch)


def _mlp_body(gg_ref, ps_ref, w1p_ref, b1_ref, w2_ref, b2_ref, out_ref):
    t = jnp.dot(ps_ref[:], w1p_ref[:],
                preferred_element_type=jnp.float32,
                precision=lax.Precision.HIGHEST)
    g3 = gg_ref[:].reshape(QB, K, 128)
    h1 = jnp.maximum(g3 - t[:, None, :] + b1_ref[:].reshape(1, 1, 128), 0.0)
    h2 = jnp.dot(h1.reshape(QB * K, 128), w2_ref[:],
                 preferred_element_type=jnp.float32,
                 precision=lax.Precision.HIGHEST) + b2_ref[:]
    out_ref[:] = jnp.max(h2.reshape(QB, K, 128), axis=1)


def _run_mlp(gg, ps_pad, W1, b1, W2, b2, interpret=False):
    w1p = jnp.concatenate(
        [W1[D:D + 3], jnp.zeros((5, 128), jnp.float32)], axis=0)
    nblk = SP // QB
    return pl.pallas_call(
        _mlp_body,
        grid=(nblk,),
        in_specs=[
            pl.BlockSpec((QB * K, 128), lambda i: (i, 0)),
            pl.BlockSpec((QB, 8), lambda i: (i, 0)),
            pl.BlockSpec((8, 128), lambda i: (0, 0)),
            pl.BlockSpec((1, 128), lambda i: (0, 0)),
            pl.BlockSpec((128, 128), lambda i: (0, 0)),
            pl.BlockSpec((1, 128), lambda i: (0, 0)),
        ],
        out_specs=pl.BlockSpec((QB, 128), lambda i: (i, 0)),
        out_shape=jax.ShapeDtypeStruct((SP, 128), jnp.float32),
        interpret=interpret,
    )(gg, ps_pad, w1p, b1.reshape(1, 128), W2, b2.reshape(1, 128))


def kernel(x, pos, batch, W1, b1, W2, b2):
    fps_idx, pos_sub = _run_fps(pos)
    g = _run_precomp(x, pos, W1)
    big = jnp.float32(1e9)
    padq = jnp.full((SP - S,), big, jnp.float32)
    psx = jnp.concatenate([pos_sub[:, 0], padq])
    psy = jnp.concatenate([pos_sub[:, 1], padq])
    psz = jnp.concatenate([pos_sub[:, 2], padq])
    padp = jnp.full((NP - N,), big, jnp.float32)
    nbr_flat = _run_nbr(jnp.concatenate([pos[:, 0], padp]),
                        jnp.concatenate([pos[:, 1], padp]),
                        jnp.concatenate([pos[:, 2], padp]),
                        psx, psy, psz)
    fidx_pad = jnp.pad(fps_idx, (0, SP - S))
    gg, bsub = _run_gather(g, nbr_flat, fidx_pad, batch)
    ps_pad = jnp.pad(pos_sub, ((0, SP - S), (0, 5)))
    out = _run_mlp(gg, ps_pad, W1, b1, W2, b2)[:S]
    return (out, pos_sub, bsub[:S])
